# baseline (speedup 1.0000x reference)
import jax
import jax.numpy as jnp
from jax import lax
from jax.experimental import pallas as pl
from jax.experimental.pallas import tpu as pltpu

B, S, D = 2, 2048, 1024
CB = 512
M = CB * CB
SUBK = 32
SLOT_DIM = 64
PK = 32
INV_C = 1.0 / float(SUBK) ** 0.5

S_CHUNK = 512
N_S_CHUNKS = S // S_CHUNK

N_M_CHUNKS = 16
M_CHUNK = M // N_M_CHUNKS
PB = M_CHUNK // CB


def _kt(s, w_ref):
    return lax.dot_general(s, w_ref[...],
                           dimension_numbers=(((1,), (1,)), ((), ())))


def _k1_body(x_ref, wa_ref, ba_ref, wb_ref, bb_ref, wv_ref, bv_ref,
             wg_ref, bg_ref, cbat_ref, cbbt_ref,
             a_ref, bv_out_ref, wupd_ref, acc_ref):
    step = pl.program_id(0)
    spb = N_S_CHUNKS // B

    @pl.when(step == 0)
    def _init():
        acc_ref[...] = jnp.zeros_like(acc_ref)

    s = jnp.sum(x_ref[...], axis=0, keepdims=True)

    @pl.when(step < spb)
    def _acc0():
        acc_ref[0:1, :] += s

    @pl.when(step >= spb)
    def _acc1():
        acc_ref[1:2, :] += s

    @pl.when(step == N_S_CHUNKS - 1)
    def _final():
        summary = acc_ref[...] * (1.0 / S)
        qa = _kt(summary, wa_ref) + ba_ref[...]
        qb = _kt(summary, wb_ref) + bb_ref[...]
        sim_a = qa @ cbat_ref[...]
        sim_b = qb @ cbbt_ref[...]

        sim = jnp.concatenate([sim_a, sim_b], axis=0)
        iota512 = lax.broadcasted_iota(jnp.int32, (2 * B, CB), 1)
        bts = lax.bitcast_convert_type(sim, jnp.int32)
        k0 = bts ^ ((bts >> 31) | jnp.int32(-2 ** 31))
        key = (k0 & jnp.int32(~(CB - 1))) | ((CB - 1) - iota512)
        neg = jnp.int32(-2 ** 31)
        for _ in range(PK):
            m = jnp.max(key, axis=1, keepdims=True)
            key = jnp.where(key == m, neg, key)
        sel = key == neg
        vmax = jnp.max(sim, axis=1, keepdims=True)
        wfull = jnp.where(sel, jnp.exp((sim - vmax) * INV_C), 0.0)
        wn = wfull / jnp.sum(wfull, axis=1, keepdims=True)
        wnt = jnp.transpose(wn)
        a_ref[...] = wnt[:, 0:B]
        bv_out_ref[...] = wnt[:, B:2 * B]

        z = jnp.sum(summary * wg_ref[...], axis=1,
                    keepdims=True) + bg_ref[...]
        gate = 1.0 / (1.0 + jnp.exp(-z))
        wupd = (0.1 * gate) * (_kt(summary, wv_ref) + bv_ref[...])
        wupd_ref[...] = wupd[:, :, None]


def _run_k1(x4, Wa, ba2, Wb, bb2, Wv, bv2, Wg, bg2, cbat, cbbt):
    whole = lambda shape: pl.BlockSpec(shape, lambda i: tuple(0 for _ in shape))
    rows = x4.shape[0] // N_S_CHUNKS
    outs = (
        jax.ShapeDtypeStruct((CB, B), jnp.float32),
        jax.ShapeDtypeStruct((CB, B), jnp.float32),
        jax.ShapeDtypeStruct((B, SLOT_DIM, 1), jnp.float32),
    )
    return pl.pallas_call(
        _k1_body,
        grid=(N_S_CHUNKS,),
        in_specs=[
            pl.BlockSpec((rows, D), lambda i: (i, 0)),
            whole(Wa.shape), whole(ba2.shape), whole(Wb.shape), whole(bb2.shape),
            whole(Wv.shape), whole(bv2.shape), whole(Wg.shape), whole(bg2.shape),
            whole(cbat.shape), whole(cbbt.shape),
        ],
        out_specs=[whole(o.shape) for o in outs],
        out_shape=outs,
        scratch_shapes=[pltpu.VMEM((B, D), jnp.float32)],
    )(x4, Wa, ba2, Wb, bb2, Wv, bv2, Wg, bg2, cbat, cbbt)


def _km_body(mt_ref, wupd_ref, wf_ref, wo_ref, bo_ref,
             out_ref, rp_ref, ro_ref):
    step = pl.program_id(0)

    @pl.when(step == 0)
    def _init():
        ro_ref[...] = jnp.zeros_like(ro_ref)

    blk = mt_ref[...]
    out_ref[...] = blk + wupd_ref[...]

    for b in range(B):
        contrib = lax.dot_general(
            blk[b], wf_ref[b:b + 1, :],
            dimension_numbers=(((1,), (1,)), ((), ())),
        )
        ro_ref[:, b:b + 1] += contrib

    @pl.when(step == N_M_CHUNKS - 1)
    def _final():
        rp = lax.dot_general(
            ro_ref[...], wo_ref[...],
            dimension_numbers=(((0,), (0,)), ((), ())),
        ) + bo_ref[...]
        rp_ref[...] = rp[:, None, :]


def _run_km(mt, wupd3, wf, Wo, bo2):
    whole = lambda shape: pl.BlockSpec(shape, lambda i: tuple(0 for _ in shape))
    outs = (
        jax.ShapeDtypeStruct((B, SLOT_DIM, M), jnp.float32),
        jax.ShapeDtypeStruct((B, 1, D), jnp.float32),
    )
    return pl.pallas_call(
        _km_body,
        grid=(N_M_CHUNKS,),
        in_specs=[
            pl.BlockSpec((B, SLOT_DIM, M_CHUNK), lambda i: (0, 0, i)),
            whole(wupd3.shape),
            pl.BlockSpec((B, M_CHUNK), lambda i: (0, i)),
            whole(Wo.shape), whole(bo2.shape),
        ],
        out_specs=[
            pl.BlockSpec((B, SLOT_DIM, M_CHUNK), lambda i: (0, 0, i)),
            whole((B, 1, D)),
        ],
        out_shape=outs,
        scratch_shapes=[pltpu.VMEM((SLOT_DIM, B), jnp.float32)],
    )(mt, wupd3, wf, Wo, bo2)


def _k4_body(big_ref, row_ref, out_ref):
    out_ref[...] = big_ref[...] + row_ref[0]


def _run_k4(big4, rows3, n_chunks):
    n, w = big4.shape
    chunk = n // n_chunks
    return pl.pallas_call(
        _k4_body,
        grid=(n_chunks,),
        in_specs=[
            pl.BlockSpec((chunk, w), lambda i: (i, 0)),
            pl.BlockSpec((1, 1, w), lambda i: (i // (n_chunks // B), 0, 0)),
        ],
        out_specs=pl.BlockSpec((chunk, w), lambda i: (i, 0)),
        out_shape=jax.ShapeDtypeStruct((n, w), jnp.float32),
    )(big4, rows3)


def kernel(x, memory, Wa, ba, Wb, bb, Wv, bv, Wo, bo, Wg, bg, codebook_a, codebook_b):
    ba2 = ba.reshape(1, SUBK)
    bb2 = bb.reshape(1, SUBK)
    bv2 = bv.reshape(1, SLOT_DIM)
    bg2 = bg.reshape(1, 1)
    bo2 = bo.reshape(1, D)
    cbat = codebook_a.T
    cbbt = codebook_b.T
    WaT, WbT, WvT, WgT = Wa.T, Wb.T, Wv.T, Wg.T

    x4 = x.reshape(B * S, D)
    At, BvT, wupd3 = _run_k1(x4, WaT, ba2, WbT, bb2, WvT, bv2, WgT, bg2,
                             cbat, cbbt)

    wf = (At.T[:, :, None] * BvT.T[:, None, :]).reshape(B, M)

    mt = jnp.transpose(memory, (0, 2, 1))
    out_t, rp = _run_km(mt, wupd3, wf, Wo, bo2)
    memory_new = jnp.transpose(out_t, (0, 2, 1))

    x_aug = _run_k4(x4, rp, 4).reshape(B, S, D)
    return (x_aug, memory_new)
